# baseline (speedup 1.0000x reference)
import functools

import jax
import jax.numpy as jnp
from jax import lax
from jax.experimental import pallas as pl
from jax.experimental.pallas import tpu as pltpu


def _sincos(x):
    ki = (x * 0.6366197723675814 + 16384.5).astype(jnp.int32) - 16384
    n = ki.astype(jnp.float32)
    r = x - n * 1.5707963705062866
    r = r - n * (-4.3711388286737929e-08)
    k = ki & 3
    r2 = r * r
    sp = (-1.9841271e-4 * r2 + 8.3333310e-3) * r2 - 1.6666667e-1
    s = r + r * r2 * sp
    cp = (-1.3888889e-3 * r2 + 4.1666668e-2) * r2 - 0.5
    c = 1.0 + r2 * cp
    swap = (k & 1) == 1
    ss = jnp.where(swap, c, s)
    cc = jnp.where(swap, s, c)
    sin_x = jnp.where((k & 2) == 2, -ss, ss)
    cos_x = jnp.where(((k + 1) & 2) == 2, -cc, cc)
    return sin_x, cos_x


def _leafmp_body(B, W, N, NB,
                 hc_ref, he_ref, ts_ref, dst_ref,
                 omega_ref, gt_ref, bg_ref, wlet_ref, ble_ref,
                 wlct_ref, blc_ref, wgt_ref, bgg_ref,
                 out_ref,
                 c_ref, acc_ref, m_ref, den_ref):
    pid = pl.program_id(0)

    @pl.when(pid == 0)
    def _init():
        c_ref[...] = jnp.zeros_like(c_ref)
        acc_ref[...] = jnp.zeros_like(acc_ref)
        m_ref[...] = jnp.full_like(m_ref, -1e30)
        den_ref[...] = jnp.zeros_like(den_ref)
        c_ref[0:N, :] = (
            jnp.dot(hc_ref[...], wlct_ref[...],
                    preferred_element_type=jnp.float32) + blc_ref[...])

    ts = ts_ref[...]
    phase = ts * omega_ref[...]
    sin_p, cos_p = _sincos(phase)
    t2v = jnp.concatenate([cos_p, sin_p], axis=1)
    gate_pre = jnp.dot(t2v, gt_ref[...],
                       preferred_element_type=jnp.float32) + bg_ref[...]
    gate = 1.0 / (1.0 + jnp.exp(-gate_pre))
    he = he_ref[...]
    proj_e = jnp.dot(he, wlet_ref[...],
                     preferred_element_type=jnp.float32) + ble_ref[...]
    q = proj_e * gate
    gh = jnp.dot(he.astype(jnp.bfloat16), wgt_ref[...].astype(jnp.bfloat16),
                 preferred_element_type=jnp.float32) + bgg_ref[...]

    dstv = dst_ref[...]
    d_lo = jnp.min(dstv)
    d_hi = jnp.max(dstv)
    nw = (d_hi - d_lo) // W + 1

    col = lax.broadcasted_iota(jnp.int32, (B, W), 1)

    def _window(w, carry):
        base = d_lo + w * W
        oh = (dstv == (base + col)).astype(jnp.float32)
        c_win = c_ref[pl.ds(base, W), :]
        cg = jnp.dot(oh, c_win, preferred_element_type=jnp.float32)
        s = jnp.sum(q * cg, axis=1, keepdims=True)
        mb = jnp.max(s)
        m_old = m_ref[pl.ds(base, W), :]
        m_new = jnp.maximum(m_old, mb)
        scale = jnp.exp(m_old - m_new)
        f = jnp.exp(mb - m_new)
        u = jnp.exp(s - mb)
        den_c = lax.dot_general(oh, u, (((0,), (0,)), ((), ())),
                                preferred_element_type=jnp.float32)
        ctr = lax.dot_general(oh.astype(jnp.bfloat16),
                              (u * gh).astype(jnp.bfloat16),
                              (((0,), (0,)), ((), ())),
                              preferred_element_type=jnp.float32)
        m_ref[pl.ds(base, W), :] = m_new
        den_ref[pl.ds(base, W), :] = den_ref[pl.ds(base, W), :] * scale + f * den_c
        acc_ref[pl.ds(base, W), :] = acc_ref[pl.ds(base, W), :] * scale + f * ctr
        return carry

    lax.fori_loop(0, nw, _window, 0)

    @pl.when(pid == NB - 1)
    def _final():
        den = den_ref[0:N, :]
        den_safe = jnp.where(den > 0.0, den, 1.0)
        agg = acc_ref[0:N, :] / den_safe
        hc = hc_ref[...]
        out_ref[...] = jnp.where(den > 0.0, 0.5 * agg + 0.5 * hc, hc)


@jax.jit
def kernel(H_e, H_c, timestamps, edge_dst, W_le, b_le, W_lc, b_lc,
           W_lt, b_lt, W_g, b_g, b_e, b_c, omega, W_t2v, b_t2v):
    E, D = H_e.shape
    N = H_c.shape[0]
    B = next(b for b in (3200, 1600, 800, 400, 160, 80, 16, 8) if E % b == 0)
    W = 128
    NPAD = -(-(N + W) // 8) * 8
    NB = E // B

    gt = W_t2v.T @ W_lt.T
    bg = (b_t2v @ W_lt.T + b_lt)[None, :]
    ble = (b_le + b_e)[None, :]
    blc = (b_lc + b_c)[None, :]
    bgg = b_g[None, :]

    ts2 = timestamps[:, None].astype(jnp.float32)
    dst2 = edge_dst[:, None].astype(jnp.int32)

    full = lambda r, c: pl.BlockSpec((r, c), lambda i: (0, 0))
    return pl.pallas_call(
        functools.partial(_leafmp_body, B, W, N, NB),
        grid=(NB,),
        in_specs=[
            full(N, D),
            pl.BlockSpec((B, D), lambda i: (i, 0)),
            pl.BlockSpec((B, 1), lambda i: (i, 0)),
            pl.BlockSpec((B, 1), lambda i: (i, 0)),
            full(1, omega.shape[0]),
            full(D, D), full(1, D),
            full(D, D), full(1, D),
            full(D, D), full(1, D),
            full(D, D), full(1, D),
        ],
        out_specs=full(N, D),
        out_shape=jax.ShapeDtypeStruct((N, D), jnp.float32),
        scratch_shapes=[
            pltpu.VMEM((NPAD, D), jnp.float32),
            pltpu.VMEM((NPAD, D), jnp.float32),
            pltpu.VMEM((NPAD, 1), jnp.float32),
            pltpu.VMEM((NPAD, 1), jnp.float32),
        ],
    )(H_c, H_e, ts2, dst2, omega[None, :], gt, bg,
      W_le.T, ble, W_lc.T, blc, W_g.T, bgg)


# speedup vs baseline: 1.0262x; 1.0262x over previous
import functools

import jax
import jax.numpy as jnp
from jax import lax
from jax.experimental import pallas as pl
from jax.experimental.pallas import tpu as pltpu


def _sincos(x):
    ki = (x * 0.6366197723675814 + 16384.5).astype(jnp.int32) - 16384
    n = ki.astype(jnp.float32)
    r = x - n * 1.5707963705062866
    r = r - n * (-4.3711388286737929e-08)
    k = ki & 3
    r2 = r * r
    sp = (-1.9841271e-4 * r2 + 8.3333310e-3) * r2 - 1.6666667e-1
    s = r + r * r2 * sp
    cp = (-1.3888889e-3 * r2 + 4.1666668e-2) * r2 - 0.5
    c = 1.0 + r2 * cp
    swap = (k & 1) == 1
    ss = jnp.where(swap, c, s)
    cc = jnp.where(swap, s, c)
    sin_x = jnp.where((k & 2) == 2, -ss, ss)
    cos_x = jnp.where(((k + 1) & 2) == 2, -cc, cc)
    return sin_x, cos_x


def _leafmp_body(B, W, N, NB,
                 hc_ref, he_ref, ts_ref, dst_ref,
                 omega_ref, gt_ref, bg_ref, wlet_ref, ble_ref,
                 wlct_ref, blc_ref, wgt_ref, bgg_ref,
                 out_ref,
                 c_ref, acc_ref, m_ref, den_ref):
    pid = pl.program_id(0)

    @pl.when(pid == 0)
    def _init():
        c_ref[...] = jnp.zeros_like(c_ref)
        acc_ref[...] = jnp.zeros_like(acc_ref)
        m_ref[...] = jnp.full_like(m_ref, -1e30)
        den_ref[...] = jnp.zeros_like(den_ref)
        c_ref[0:N, :] = (
            jnp.dot(hc_ref[...], wlct_ref[...],
                    preferred_element_type=jnp.float32) + blc_ref[...])

    ts = ts_ref[...]
    phase = ts * omega_ref[...]
    sin_p, cos_p = _sincos(phase)
    t2v = jnp.concatenate([cos_p, sin_p], axis=1)
    gate_pre = jnp.dot(t2v, gt_ref[...],
                       preferred_element_type=jnp.float32) + bg_ref[...]
    gate = 1.0 / (1.0 + jnp.exp(-gate_pre))
    he = he_ref[...]
    proj_e = jnp.dot(he, wlet_ref[...],
                     preferred_element_type=jnp.float32) + ble_ref[...]
    q = proj_e * gate
    gh = jnp.dot(he, wgt_ref[...],
                 preferred_element_type=jnp.float32) + bgg_ref[...]

    dstv = dst_ref[...]
    d_lo = jnp.min(dstv)
    d_hi = jnp.max(dstv)
    nw = (d_hi - d_lo) // W + 1

    col = lax.broadcasted_iota(jnp.int32, (B, W), 1)
    ones_col = jnp.ones((q.shape[1], 1), jnp.float32)

    def _window(w, carry):
        base = d_lo + w * W
        oh = (dstv == (base + col)).astype(jnp.float32)
        c_win = c_ref[pl.ds(base, W), :]
        cg = jnp.dot(oh, c_win, preferred_element_type=jnp.float32)
        s = jnp.dot(q * cg, ones_col,
                    preferred_element_type=jnp.float32)
        mb = jnp.max(s)
        m_old = m_ref[pl.ds(base, W), :]
        m_new = jnp.maximum(m_old, mb)
        scale = jnp.exp(m_old - m_new)
        f = jnp.exp(mb - m_new)
        u = jnp.exp(s - mb)
        den_c = lax.dot_general(oh, u, (((0,), (0,)), ((), ())),
                                preferred_element_type=jnp.float32)
        ctr = lax.dot_general(oh, u * gh, (((0,), (0,)), ((), ())),
                              preferred_element_type=jnp.float32)
        m_ref[pl.ds(base, W), :] = m_new
        den_ref[pl.ds(base, W), :] = den_ref[pl.ds(base, W), :] * scale + f * den_c
        acc_ref[pl.ds(base, W), :] = acc_ref[pl.ds(base, W), :] * scale + f * ctr
        return carry

    lax.fori_loop(0, nw, _window, 0)

    @pl.when(pid == NB - 1)
    def _final():
        den = den_ref[0:N, :]
        den_safe = jnp.where(den > 0.0, den, 1.0)
        agg = acc_ref[0:N, :] / den_safe
        hc = hc_ref[...]
        out_ref[...] = jnp.where(den > 0.0, 0.5 * agg + 0.5 * hc, hc)


@jax.jit
def kernel(H_e, H_c, timestamps, edge_dst, W_le, b_le, W_lc, b_lc,
           W_lt, b_lt, W_g, b_g, b_e, b_c, omega, W_t2v, b_t2v):
    E, D = H_e.shape
    N = H_c.shape[0]
    B = next(b for b in (3200, 1600, 800, 400, 160, 80, 16, 8) if E % b == 0)
    W = 128
    NPAD = -(-(N + W) // 8) * 8
    NB = E // B

    gt = W_t2v.T @ W_lt.T
    bg = (b_t2v @ W_lt.T + b_lt)[None, :]
    ble = (b_le + b_e)[None, :]
    blc = (b_lc + b_c)[None, :]
    bgg = b_g[None, :]

    ts2 = timestamps[:, None].astype(jnp.float32)
    dst2 = edge_dst[:, None].astype(jnp.int32)

    full = lambda r, c: pl.BlockSpec((r, c), lambda i: (0, 0))
    return pl.pallas_call(
        functools.partial(_leafmp_body, B, W, N, NB),
        grid=(NB,),
        in_specs=[
            full(N, D),
            pl.BlockSpec((B, D), lambda i: (i, 0)),
            pl.BlockSpec((B, 1), lambda i: (i, 0)),
            pl.BlockSpec((B, 1), lambda i: (i, 0)),
            full(1, omega.shape[0]),
            full(D, D), full(1, D),
            full(D, D), full(1, D),
            full(D, D), full(1, D),
            full(D, D), full(1, D),
        ],
        out_specs=full(N, D),
        out_shape=jax.ShapeDtypeStruct((N, D), jnp.float32),
        scratch_shapes=[
            pltpu.VMEM((NPAD, D), jnp.float32),
            pltpu.VMEM((NPAD, D), jnp.float32),
            pltpu.VMEM((NPAD, 1), jnp.float32),
            pltpu.VMEM((NPAD, 1), jnp.float32),
        ],
    )(H_c, H_e, ts2, dst2, omega[None, :], gt, bg,
      W_le.T, ble, W_lc.T, blc, W_g.T, bgg)


# speedup vs baseline: 1.0760x; 1.0485x over previous
import functools

import jax
import jax.numpy as jnp
from jax import lax
from jax.experimental import pallas as pl
from jax.experimental.pallas import tpu as pltpu


def _sincos(x):
    ki = (x * 0.6366197723675814 + 16384.5).astype(jnp.int32) - 16384
    n = ki.astype(jnp.float32)
    r = x - n * 1.5707963705062866
    r = r - n * (-4.3711388286737929e-08)
    k = ki & 3
    r2 = r * r
    sp = (-1.9841271e-4 * r2 + 8.3333310e-3) * r2 - 1.6666667e-1
    s = r + r * r2 * sp
    cp = (-1.3888889e-3 * r2 + 4.1666668e-2) * r2 - 0.5
    c = 1.0 + r2 * cp
    swap = (k & 1) == 1
    ss = jnp.where(swap, c, s)
    cc = jnp.where(swap, s, c)
    sin_x = jnp.where((k & 2) == 2, -ss, ss)
    cos_x = jnp.where(((k + 1) & 2) == 2, -cc, cc)
    return sin_x, cos_x


def _leafmp_body(B, W, N, NB,
                 bounds_ref,
                 hc_ref, he_ref, ts_ref, dst_ref,
                 omega_ref, gt_ref, bg_ref, wlet_ref, ble_ref,
                 wlct_ref, blc_ref, wgt_ref, bgg_ref,
                 out_ref,
                 c_ref, acc_ref, m_ref, den_ref):
    pid = pl.program_id(0)

    @pl.when(pid == 0)
    def _init():
        c_ref[...] = jnp.zeros_like(c_ref)
        acc_ref[...] = jnp.zeros_like(acc_ref)
        m_ref[...] = jnp.full_like(m_ref, -1e30)
        den_ref[...] = jnp.zeros_like(den_ref)
        c_ref[0:N, :] = (
            jnp.dot(hc_ref[...], wlct_ref[...],
                    preferred_element_type=jnp.float32) + blc_ref[...])

    ts = ts_ref[...]
    phase = ts * omega_ref[...]
    sin_p, cos_p = _sincos(phase)
    t2v = jnp.concatenate([cos_p, sin_p], axis=1)
    gate_pre = jnp.dot(t2v, gt_ref[...],
                       preferred_element_type=jnp.float32) + bg_ref[...]
    gate = 1.0 / (1.0 + jnp.exp(-gate_pre))
    he = he_ref[...]
    proj_e = jnp.dot(he, wlet_ref[...],
                     preferred_element_type=jnp.float32) + ble_ref[...]
    q = proj_e * gate
    gh = jnp.dot(he, wgt_ref[...],
                 preferred_element_type=jnp.float32) + bgg_ref[...]

    dstv = dst_ref[...]
    d_lo = bounds_ref[0, pid]
    d_hi = bounds_ref[1, pid]
    nw = (d_hi - d_lo) // W + 1

    col = lax.broadcasted_iota(jnp.int32, (B, W), 1)
    ones_col = jnp.ones((q.shape[1], 1), jnp.float32)

    def _window(w, carry):
        base = d_lo + w * W
        oh = (dstv == (base + col)).astype(jnp.float32)
        c_win = c_ref[pl.ds(base, W), :]
        cg = jnp.dot(oh, c_win, preferred_element_type=jnp.float32)
        s = jnp.dot(q * cg, ones_col,
                    preferred_element_type=jnp.float32)
        mb = jnp.max(s)
        m_old = m_ref[pl.ds(base, W), :]
        m_new = jnp.maximum(m_old, mb)
        scale = jnp.exp(m_old - m_new)
        f = jnp.exp(mb - m_new)
        u = jnp.exp(s - mb)
        den_c = lax.dot_general(oh, u, (((0,), (0,)), ((), ())),
                                preferred_element_type=jnp.float32)
        ctr = lax.dot_general(oh, u * gh, (((0,), (0,)), ((), ())),
                              preferred_element_type=jnp.float32)
        m_ref[pl.ds(base, W), :] = m_new
        den_ref[pl.ds(base, W), :] = den_ref[pl.ds(base, W), :] * scale + f * den_c
        acc_ref[pl.ds(base, W), :] = acc_ref[pl.ds(base, W), :] * scale + f * ctr
        return carry

    lax.fori_loop(0, nw, _window, 0)

    @pl.when(pid == NB - 1)
    def _final():
        den = den_ref[0:N, :]
        den_safe = jnp.where(den > 0.0, den, 1.0)
        agg = acc_ref[0:N, :] / den_safe
        hc = hc_ref[...]
        out_ref[...] = jnp.where(den > 0.0, 0.5 * agg + 0.5 * hc, hc)


@jax.jit
def kernel(H_e, H_c, timestamps, edge_dst, W_le, b_le, W_lc, b_lc,
           W_lt, b_lt, W_g, b_g, b_e, b_c, omega, W_t2v, b_t2v):
    E, D = H_e.shape
    N = H_c.shape[0]
    B = next(b for b in (3200, 1600, 800, 400, 160, 80, 16, 8) if E % b == 0)
    W = 128
    NPAD = -(-(N + W) // 8) * 8
    NB = E // B

    gt = W_t2v.T @ W_lt.T
    bg = (b_t2v @ W_lt.T + b_lt)[None, :]
    ble = (b_le + b_e)[None, :]
    blc = (b_lc + b_c)[None, :]
    bgg = b_g[None, :]

    ts2 = timestamps[:, None].astype(jnp.float32)
    dst2 = edge_dst[:, None].astype(jnp.int32)

    bounds = jnp.stack([edge_dst[0::B], edge_dst[B - 1::B]]).astype(jnp.int32)

    full = lambda r, c: pl.BlockSpec((r, c), lambda i, *_: (0, 0))
    grid_spec = pltpu.PrefetchScalarGridSpec(
        num_scalar_prefetch=1,
        grid=(NB,),
        in_specs=[
            full(N, D),
            pl.BlockSpec((B, D), lambda i, *_: (i, 0)),
            pl.BlockSpec((B, 1), lambda i, *_: (i, 0)),
            pl.BlockSpec((B, 1), lambda i, *_: (i, 0)),
            full(1, omega.shape[0]),
            full(D, D), full(1, D),
            full(D, D), full(1, D),
            full(D, D), full(1, D),
            full(D, D), full(1, D),
        ],
        out_specs=full(N, D),
        scratch_shapes=[
            pltpu.VMEM((NPAD, D), jnp.float32),
            pltpu.VMEM((NPAD, D), jnp.float32),
            pltpu.VMEM((NPAD, 1), jnp.float32),
            pltpu.VMEM((NPAD, 1), jnp.float32),
        ],
    )
    return pl.pallas_call(
        functools.partial(_leafmp_body, B, W, N, NB),
        grid_spec=grid_spec,
        out_shape=jax.ShapeDtypeStruct((N, D), jnp.float32),
    )(bounds, H_c, H_e, ts2, dst2, omega[None, :], gt, bg,
      W_le.T, ble, W_lc.T, blc, W_g.T, bgg)


# speedup vs baseline: 1.0859x; 1.0092x over previous
import functools

import jax
import jax.numpy as jnp
from jax import lax
from jax.experimental import pallas as pl
from jax.experimental.pallas import tpu as pltpu


def _sincos(x):
    ki = (x * 0.6366197723675814 + 16384.5).astype(jnp.int32) - 16384
    n = ki.astype(jnp.float32)
    r = x - n * 1.5707963705062866
    r = r - n * (-4.3711388286737929e-08)
    k = ki & 3
    r2 = r * r
    sp = (-1.9841271e-4 * r2 + 8.3333310e-3) * r2 - 1.6666667e-1
    s = r + r * r2 * sp
    cp = (-1.3888889e-3 * r2 + 4.1666668e-2) * r2 - 0.5
    c = 1.0 + r2 * cp
    swap = (k & 1) == 1
    ss = jnp.where(swap, c, s)
    cc = jnp.where(swap, s, c)
    sin_x = jnp.where((k & 2) == 2, -ss, ss)
    cos_x = jnp.where(((k + 1) & 2) == 2, -cc, cc)
    return sin_x, cos_x


def _leafmp_body(B, W, N, NB,
                 bounds_ref,
                 hc_ref, he_ref, ts_ref, dst_ref,
                 omega_ref, gt_ref, bg_ref, wlet_ref, ble_ref,
                 wlct_ref, blc_ref, bgg_ref,
                 out_ref,
                 c_ref, acc_ref, m_ref, den_ref):
    pid = pl.program_id(0)

    @pl.when(pid == 0)
    def _init():
        c_ref[...] = jnp.zeros_like(c_ref)
        acc_ref[...] = jnp.zeros_like(acc_ref)
        m_ref[...] = jnp.full_like(m_ref, -1e30)
        den_ref[...] = jnp.zeros_like(den_ref)
        c_ref[0:N, :] = (
            jnp.dot(hc_ref[...], wlct_ref[...],
                    preferred_element_type=jnp.float32) + blc_ref[...])

    ts = ts_ref[...]
    phase = ts * omega_ref[...]
    sin_p, cos_p = _sincos(phase)
    t2v = jnp.concatenate([cos_p, sin_p], axis=1)
    gate_pre = jnp.dot(t2v, gt_ref[...],
                       preferred_element_type=jnp.float32) + bg_ref[...]
    gate = 1.0 / (1.0 + jnp.exp(-gate_pre))
    he = he_ref[...]
    pw = jnp.dot(he, wlet_ref[...],
                 preferred_element_type=jnp.float32)
    D = pw.shape[1] // 2
    proj_e = pw[:, :D] + ble_ref[...]
    q = proj_e * gate
    gh = pw[:, D:] + bgg_ref[...]

    dstv = dst_ref[...]
    d_lo = bounds_ref[0, pid]
    d_hi = bounds_ref[1, pid]
    nw = (d_hi - d_lo) // W + 1

    col = lax.broadcasted_iota(jnp.int32, (B, W), 1)
    ones_col = jnp.ones((q.shape[1], 1), jnp.float32)

    def _window(w, carry):
        base = d_lo + w * W
        oh = (dstv == (base + col)).astype(jnp.float32)
        c_win = c_ref[pl.ds(base, W), :]
        cg = jnp.dot(oh, c_win, preferred_element_type=jnp.float32)
        s = jnp.dot(q * cg, ones_col,
                    preferred_element_type=jnp.float32)
        mb = jnp.max(s)
        m_old = m_ref[pl.ds(base, W), :]
        m_new = jnp.maximum(m_old, mb)
        scale = jnp.exp(m_old - m_new)
        f = jnp.exp(mb - m_new)
        u = jnp.exp(s - mb)
        den_c = lax.dot_general(oh, u, (((0,), (0,)), ((), ())),
                                preferred_element_type=jnp.float32)
        ctr = lax.dot_general(oh, u * gh, (((0,), (0,)), ((), ())),
                              preferred_element_type=jnp.float32)
        m_ref[pl.ds(base, W), :] = m_new
        den_ref[pl.ds(base, W), :] = den_ref[pl.ds(base, W), :] * scale + f * den_c
        acc_ref[pl.ds(base, W), :] = acc_ref[pl.ds(base, W), :] * scale + f * ctr
        return carry

    lax.fori_loop(0, nw, _window, 0)

    @pl.when(pid == NB - 1)
    def _final():
        den = den_ref[0:N, :]
        den_safe = jnp.where(den > 0.0, den, 1.0)
        agg = acc_ref[0:N, :] / den_safe
        hc = hc_ref[...]
        out_ref[...] = jnp.where(den > 0.0, 0.5 * agg + 0.5 * hc, hc)


@jax.jit
def kernel(H_e, H_c, timestamps, edge_dst, W_le, b_le, W_lc, b_lc,
           W_lt, b_lt, W_g, b_g, b_e, b_c, omega, W_t2v, b_t2v):
    E, D = H_e.shape
    N = H_c.shape[0]
    B = next(b for b in (3200, 1600, 800, 400, 160, 80, 16, 8) if E % b == 0)
    W = 128
    NPAD = -(-(N + W) // 8) * 8
    NB = E // B

    gt = W_t2v.T @ W_lt.T
    bg = (b_t2v @ W_lt.T + b_lt)[None, :]
    ble = (b_le + b_e)[None, :]
    blc = (b_lc + b_c)[None, :]
    bgg = b_g[None, :]

    ts2 = timestamps[:, None].astype(jnp.float32)
    dst2 = edge_dst[:, None].astype(jnp.int32)

    bounds = jnp.stack([edge_dst[0::B], edge_dst[B - 1::B]]).astype(jnp.int32)

    full = lambda r, c: pl.BlockSpec((r, c), lambda i, *_: (0, 0))
    grid_spec = pltpu.PrefetchScalarGridSpec(
        num_scalar_prefetch=1,
        grid=(NB,),
        in_specs=[
            full(N, D),
            pl.BlockSpec((B, D), lambda i, *_: (i, 0)),
            pl.BlockSpec((B, 1), lambda i, *_: (i, 0)),
            pl.BlockSpec((B, 1), lambda i, *_: (i, 0)),
            full(1, omega.shape[0]),
            full(D, D), full(1, D),
            full(D, 2 * D), full(1, D),
            full(D, D), full(1, D),
            full(1, D),
        ],
        out_specs=full(N, D),
        scratch_shapes=[
            pltpu.VMEM((NPAD, D), jnp.float32),
            pltpu.VMEM((NPAD, D), jnp.float32),
            pltpu.VMEM((NPAD, 1), jnp.float32),
            pltpu.VMEM((NPAD, 1), jnp.float32),
        ],
    )
    return pl.pallas_call(
        functools.partial(_leafmp_body, B, W, N, NB),
        grid_spec=grid_spec,
        out_shape=jax.ShapeDtypeStruct((N, D), jnp.float32),
    )(bounds, H_c, H_e, ts2, dst2, omega[None, :], gt, bg,
      jnp.concatenate([W_le.T, W_g.T], axis=1), ble, W_lc.T, blc, bgg)
